# baseline (speedup 1.0000x reference)
import functools

import jax
import jax.numpy as jnp
from jax import lax
from jax.experimental import pallas as pl
from jax.experimental.pallas import tpu as pltpu
from jax.experimental.pallas import tpu_sc as plsc

N = 10000
N_PAD = 10240
D_IN = 128
D_HID = 256
D_OUT = 128
E = 320000

CHUNK = 40
NCH = 256
NB = 8
GLEAD = 7
E_TILE = CHUNK * NCH
E_SC = E_TILE * 16
E_PAD = E_SC * 2

DCHUNK = 128
DNCH = E_PAD // 32 // DCHUNK
ROWS_TILE = N_PAD // 16
NWB = ROWS_TILE // CHUNK

_MESH = plsc.VectorSubcoreMesh(core_axis_name="c", subcore_axis_name="s")


@functools.partial(
    pl.kernel,
    out_type=jax.ShapeDtypeStruct((2, N_PAD), jnp.float32),
    mesh=_MESH,
    scratch_types=[
        pltpu.VMEM((DNCH, DCHUNK), jnp.int32),
        pltpu.VMEM((DCHUNK,), jnp.float32),
        pltpu.VMEM((ROWS_TILE,), jnp.float32),
        pltpu.VMEM_SHARED((N_PAD,), jnp.float32),
        pltpu.SemaphoreType.DMA((2,)),
    ],
)
def _deg_kernel(dst2d_hbm, out_hbm, di_v, ones_v, buf_v, deg_sh, sems):
    c = lax.axis_index("c")
    s = lax.axis_index("s")
    row0 = s * ROWS_TILE
    tile = c * 16 + s

    pltpu.sync_copy(dst2d_hbm.at[pl.ds(tile * DNCH, DNCH)], di_v)

    def fill_ones(i, _):
        ones_v[pl.ds(i * 16, 16)] = jnp.full((16,), 1.0, jnp.float32)
        return 0

    lax.fori_loop(0, DCHUNK // 16, fill_ones, 0)

    def fill_zero(i, _):
        buf_v[pl.ds(i * 16, 16)] = jnp.zeros((16,), jnp.float32)
        return 0

    lax.fori_loop(0, ROWS_TILE // 16, fill_zero, 0)

    pltpu.sync_copy(buf_v, deg_sh.at[pl.ds(row0, ROWS_TILE)])
    plsc.subcore_barrier()

    WAVE = 8
    for w in range(DNCH // WAVE):
        sem = sems.at[w % 2]
        if w >= 2:
            for i in range(WAVE):
                pltpu.make_async_copy(
                    ones_v, deg_sh.at[di_v.at[(w - 2) * WAVE + i]], sem
                ).wait()
        for i in range(WAVE):
            pltpu.async_copy(
                ones_v, deg_sh.at[di_v.at[w * WAVE + i]], sem, add=True)
    for w in (DNCH // WAVE - 2, DNCH // WAVE - 1):
        sem = sems.at[w % 2]
        for i in range(WAVE):
            pltpu.make_async_copy(
                ones_v, deg_sh.at[di_v.at[w * WAVE + i]], sem
            ).wait()

    plsc.subcore_barrier()
    pltpu.sync_copy(deg_sh.at[pl.ds(row0, ROWS_TILE)], buf_v)
    pltpu.sync_copy(buf_v, out_hbm.at[c, pl.ds(row0, ROWS_TILE)])


@functools.partial(
    pl.kernel,
    out_type=jax.ShapeDtypeStruct((2, N_PAD, D_IN), jnp.float32),
    mesh=_MESH,
    scratch_types=[
        [pltpu.VMEM((NB, CHUNK), jnp.int32) for _ in range(2)],
        [pltpu.VMEM((NB, CHUNK), jnp.int32) for _ in range(2)],
        [pltpu.VMEM((CHUNK, D_IN), jnp.float32) for _ in range(NB)],
        pltpu.VMEM_SHARED((N_PAD, D_IN), jnp.float32),
        pltpu.SemaphoreType.DMA((NB,)),
        pltpu.SemaphoreType.DMA((NB,)),
        pltpu.SemaphoreType.DMA((2,)),
        pltpu.SemaphoreType.DMA((2,)),
    ],
)
def _prop_kernel(xs_hbm, src2d_hbm, dst2d_hbm, out_hbm, si_h, di_h, rows,
                 acc_sh, gsem, ssem, sisem, disem):
    c = lax.axis_index("c")
    s = lax.axis_index("s")
    row0 = s * ROWS_TILE
    cbase = (c * 16 + s) * NCH

    def fire_gather(b, half, r):
        return pltpu.async_copy(
            xs_hbm.at[si_h[half].at[r]], rows[b], gsem.at[b])

    def wait_gather(b):
        pltpu.make_async_copy(
            xs_hbm.at[si_h[0].at[0]], rows[b], gsem.at[b]).wait()

    def fire_scatter(b, half):
        return pltpu.async_copy(
            rows[b], acc_sh.at[di_h[half].at[b]], ssem.at[b], add=True)

    def wait_scatter(b):
        pltpu.make_async_copy(
            rows[b], acc_sh.at[di_h[0].at[0]], ssem.at[b]).wait()

    pltpu.sync_copy(src2d_hbm.at[pl.ds(cbase, NB)], si_h[0])
    pltpu.sync_copy(dst2d_hbm.at[pl.ds(cbase, NB)], di_h[0])
    pltpu.async_copy(src2d_hbm.at[pl.ds(cbase + NB, NB)], si_h[1],
                     sisem.at[1])
    pltpu.async_copy(dst2d_hbm.at[pl.ds(cbase + NB, NB)], di_h[1],
                     disem.at[1])
    for k in range(min(GLEAD, NB - 2)):
        fire_gather(k, 0, k)

    @pl.when(c == 0)
    def _():
        descs = [None, None]
        for j in range(2):
            descs[j] = pltpu.async_copy(
                xs_hbm.at[pl.ds(row0 + j * CHUNK, CHUNK)],
                rows[NB - 2 + j], gsem.at[NB - 2 + j])
        for j in range(NWB):
            b = NB - 2 + (j % 2)
            descs[j % 2].wait()
            pltpu.sync_copy(
                rows[b], acc_sh.at[pl.ds(row0 + j * CHUNK, CHUNK)])
            if j + 2 < NWB:
                descs[j % 2] = pltpu.async_copy(
                    xs_hbm.at[pl.ds(row0 + (j + 2) * CHUNK, CHUNK)],
                    rows[b], gsem.at[b])

    @pl.when(c == 1)
    def _():
        def zero_row(i, _):
            def zero_seg(j, _):
                rows[NB - 2][i, pl.ds(j * 16, 16)] = jnp.zeros(
                    (16,), jnp.float32)
                return 0
            lax.fori_loop(0, D_IN // 16, zero_seg, 0)
            return 0

        lax.fori_loop(0, CHUNK, zero_row, 0)
        for j in range(NWB):
            pltpu.sync_copy(
                rows[NB - 2], acc_sh.at[pl.ds(row0 + j * CHUNK, CHUNK)])

    for k in range(NB - 2, GLEAD):
        fire_gather(k, 0, k)

    plsc.subcore_barrier()

    def group_body(g, hc, first=False, last=False):
        hn = 1 - hc
        for b in range(NB):
            if b == 0 and not first:
                pltpu.make_async_copy(
                    dst2d_hbm.at[pl.ds(cbase, NB)], di_h[hc], disem.at[hc]
                ).wait()
            if b == 0 and not first and not last:
                pltpu.async_copy(
                    src2d_hbm.at[pl.ds(cbase + (g + 1) * NB, NB)],
                    si_h[hn], sisem.at[hn])
            wait_gather(b)
            fire_scatter(b, hc)
            if not (first and b < NB - GLEAD):
                wait_scatter((b + GLEAD) % NB)
            if b == 1 and not first and not last:
                pltpu.async_copy(
                    dst2d_hbm.at[pl.ds(cbase + (g + 1) * NB, NB)],
                    di_h[hn], disem.at[hn])
            if b == NB - GLEAD and not last:
                pltpu.make_async_copy(
                    src2d_hbm.at[pl.ds(cbase, NB)], si_h[hn], sisem.at[hn]
                ).wait()
            if not last or b < NB - GLEAD:
                tgt_half = hc if b < NB - GLEAD else hn
                fire_gather((b + GLEAD) % NB, tgt_half, (b + GLEAD) % NB)
        if last:
            for b in range(GLEAD, NB):
                wait_scatter(b)
        return None

    group_body(0, 0, first=True)
    group_body(1, 1)

    def pair(gp, _):
        group_body(gp * 2, 0)
        group_body(gp * 2 + 1, 1)
        return 0

    lax.fori_loop(1, NCH // NB // 2 - 1, pair, 0)

    group_body(NCH // NB - 2, 0)
    group_body(NCH // NB - 1, 1, last=True)

    plsc.subcore_barrier()

    for j in range(NWB):
        b = j % NB
        if j >= NB:
            pltpu.make_async_copy(
                rows[b], out_hbm.at[c, pl.ds(row0 + (j - NB) * CHUNK, CHUNK)],
                gsem.at[b]).wait()
        sl = pl.ds(row0 + j * CHUNK, CHUNK)
        pltpu.sync_copy(acc_sh.at[sl], rows[b])
        pltpu.async_copy(rows[b], out_hbm.at[c, sl], gsem.at[b])
    for j in range(max(0, NWB - NB), NWB):
        b = j % NB
        pltpu.make_async_copy(
            rows[b], out_hbm.at[c, pl.ds(row0 + j * CHUNK, CHUNK)],
            gsem.at[b]).wait()


_BM = 640
_BS = 400


def _scale_body(deg_ref, x_ref, xs_ref, dinv_ref):
    d = deg_ref[...]
    dinv = lax.rsqrt(d[0] + d[1] + 1.0)
    xs_ref[...] = x_ref[...] * dinv
    dinv_ref[...] = dinv


def _scale(deg3, x):
    return pl.pallas_call(
        _scale_body,
        grid=(N // _BS,),
        in_specs=[
            pl.BlockSpec((2, _BS, 1), lambda i: (0, i, 0)),
            pl.BlockSpec((_BS, D_IN), lambda i: (i, 0)),
        ],
        out_specs=[
            pl.BlockSpec((_BS, D_IN), lambda i: (i, 0)),
            pl.BlockSpec((_BS, 1), lambda i: (i, 0)),
        ],
        out_shape=[
            jax.ShapeDtypeStruct((N_PAD, D_IN), jnp.float32),
            jax.ShapeDtypeStruct((N_PAD, 1), jnp.float32),
        ],
    )(deg3, x)


def _dense_body(agg_ref, dinv_ref, w1_ref, b1_ref, w2_ref, out_ref):
    a = agg_ref[...]
    dinv = dinv_ref[...]
    y = (a[0] + a[1]) * dinv
    h = jnp.maximum(
        jnp.dot(y, w1_ref[...], preferred_element_type=jnp.float32)
        + b1_ref[...], 0.0)
    g = jnp.dot(h, w2_ref[...], preferred_element_type=jnp.float32)
    out_ref[...] = g * dinv


def _dense(agg, dinv, W1, b1, W2):
    return pl.pallas_call(
        _dense_body,
        grid=(N // _BS,),
        in_specs=[
            pl.BlockSpec((2, _BS, D_IN), lambda i: (0, i, 0)),
            pl.BlockSpec((_BS, 1), lambda i: (i, 0)),
            pl.BlockSpec((D_IN, D_HID), lambda i: (0, 0)),
            pl.BlockSpec((1, D_HID), lambda i: (0, 0)),
            pl.BlockSpec((D_HID, D_OUT), lambda i: (0, 0)),
        ],
        out_specs=pl.BlockSpec((_BS, D_OUT), lambda i: (i, 0)),
        out_shape=jax.ShapeDtypeStruct((N_PAD, D_OUT), jnp.float32),
    )(agg, dinv, W1, b1, W2)


def _final_body(agg_ref, dinv_ref, b2_ref, out_ref):
    a = agg_ref[...]
    out_ref[...] = (a[0] + a[1]) * dinv_ref[...] + b2_ref[...]


def _final(agg, dinv, b2):
    return pl.pallas_call(
        _final_body,
        grid=(N // _BS,),
        in_specs=[
            pl.BlockSpec((2, _BS, D_OUT), lambda i: (0, i, 0)),
            pl.BlockSpec((_BS, 1), lambda i: (i, 0)),
            pl.BlockSpec((1, D_OUT), lambda i: (0, 0)),
        ],
        out_specs=pl.BlockSpec((_BS, D_OUT), lambda i: (i, 0)),
        out_shape=jax.ShapeDtypeStruct((N, D_OUT), jnp.float32),
    )(agg, dinv, b2)


def kernel(x, edge_index, W1, b1, W2, b2):
    ei = edge_index.astype(jnp.int32)
    n_dummy = E_PAD - E
    pad_idx = N + (jnp.arange(n_dummy, dtype=jnp.int32) % (N_PAD - N))
    src_p = jnp.concatenate([ei[0], pad_idx])
    dst_p = jnp.concatenate([ei[1], pad_idx])
    src2d = src_p.reshape(E_PAD // CHUNK, CHUNK)
    dst2d = dst_p.reshape(E_PAD // CHUNK, CHUNK)
    dst2d_deg = dst_p.reshape(E_PAD // DCHUNK, DCHUNK)

    deg = _deg_kernel(dst2d_deg)
    deg3 = deg.reshape(2, N_PAD, 1)
    xs, dinv = _scale(deg3, x)
    agg1 = _prop_kernel(xs, src2d, dst2d)
    gs = _dense(agg1, dinv, W1, b1.reshape(1, D_HID), W2)
    agg2 = _prop_kernel(gs, src2d, dst2d)
    return _final(agg2, dinv, b2.reshape(1, D_OUT))


# speedup vs baseline: 1.0162x; 1.0162x over previous
import functools

import jax
import jax.numpy as jnp
from jax import lax
from jax.experimental import pallas as pl
from jax.experimental.pallas import tpu as pltpu
from jax.experimental.pallas import tpu_sc as plsc

N = 10000
N_PAD = 10240
D_IN = 128
D_HID = 256
D_OUT = 128
E = 320000

CHUNK = 40
NCH = 256
NB = 8
GLEAD = 6
E_TILE = CHUNK * NCH
E_SC = E_TILE * 16
E_PAD = E_SC * 2

DCHUNK = 128
DNCH = E_PAD // 32 // DCHUNK
ROWS_TILE = N_PAD // 16
NWB = ROWS_TILE // CHUNK

_MESH = plsc.VectorSubcoreMesh(core_axis_name="c", subcore_axis_name="s")


@functools.partial(
    pl.kernel,
    out_type=jax.ShapeDtypeStruct((2, N_PAD), jnp.float32),
    mesh=_MESH,
    scratch_types=[
        pltpu.VMEM((DNCH, DCHUNK), jnp.int32),
        pltpu.VMEM((DCHUNK,), jnp.float32),
        pltpu.VMEM((ROWS_TILE,), jnp.float32),
        pltpu.VMEM_SHARED((N_PAD,), jnp.float32),
        pltpu.SemaphoreType.DMA((2,)),
    ],
)
def _deg_kernel(dst2d_hbm, out_hbm, di_v, ones_v, buf_v, deg_sh, sems):
    c = lax.axis_index("c")
    s = lax.axis_index("s")
    row0 = s * ROWS_TILE
    tile = c * 16 + s

    pltpu.sync_copy(dst2d_hbm.at[pl.ds(tile * DNCH, DNCH)], di_v)

    def fill_ones(i, _):
        ones_v[pl.ds(i * 16, 16)] = jnp.full((16,), 1.0, jnp.float32)
        return 0

    lax.fori_loop(0, DCHUNK // 16, fill_ones, 0)

    def fill_zero(i, _):
        buf_v[pl.ds(i * 16, 16)] = jnp.zeros((16,), jnp.float32)
        return 0

    lax.fori_loop(0, ROWS_TILE // 16, fill_zero, 0)

    pltpu.sync_copy(buf_v, deg_sh.at[pl.ds(row0, ROWS_TILE)])
    plsc.subcore_barrier()

    WAVE = 8
    for w in range(DNCH // WAVE):
        sem = sems.at[w % 2]
        if w >= 2:
            for i in range(WAVE):
                pltpu.make_async_copy(
                    ones_v, deg_sh.at[di_v.at[(w - 2) * WAVE + i]], sem
                ).wait()
        for i in range(WAVE):
            pltpu.async_copy(
                ones_v, deg_sh.at[di_v.at[w * WAVE + i]], sem, add=True)
    for w in (DNCH // WAVE - 2, DNCH // WAVE - 1):
        sem = sems.at[w % 2]
        for i in range(WAVE):
            pltpu.make_async_copy(
                ones_v, deg_sh.at[di_v.at[w * WAVE + i]], sem
            ).wait()

    plsc.subcore_barrier()
    pltpu.sync_copy(deg_sh.at[pl.ds(row0, ROWS_TILE)], buf_v)
    pltpu.sync_copy(buf_v, out_hbm.at[c, pl.ds(row0, ROWS_TILE)])


@functools.partial(
    pl.kernel,
    out_type=jax.ShapeDtypeStruct((2, N_PAD, D_IN), jnp.float32),
    mesh=_MESH,
    scratch_types=[
        [pltpu.VMEM((NB, CHUNK), jnp.int32) for _ in range(2)],
        [pltpu.VMEM((NB, CHUNK), jnp.int32) for _ in range(2)],
        [pltpu.VMEM((CHUNK, D_IN), jnp.float32) for _ in range(NB)],
        pltpu.VMEM_SHARED((N_PAD, D_IN), jnp.float32),
        pltpu.SemaphoreType.DMA((NB,)),
        pltpu.SemaphoreType.DMA((NB,)),
        pltpu.SemaphoreType.DMA((2,)),
        pltpu.SemaphoreType.DMA((2,)),
    ],
)
def _prop_kernel(xs_hbm, src2d_hbm, dst2d_hbm, out_hbm, si_h, di_h, rows,
                 acc_sh, gsem, ssem, sisem, disem):
    c = lax.axis_index("c")
    s = lax.axis_index("s")
    row0 = s * ROWS_TILE
    cbase = (c * 16 + s) * NCH

    def fire_gather(b, half, r):
        return pltpu.async_copy(
            xs_hbm.at[si_h[half].at[r]], rows[b], gsem.at[b])

    def wait_gather(b):
        pltpu.make_async_copy(
            xs_hbm.at[si_h[0].at[0]], rows[b], gsem.at[b]).wait()

    def fire_scatter(b, half):
        return pltpu.async_copy(
            rows[b], acc_sh.at[di_h[half].at[b]], ssem.at[b], add=True)

    def wait_scatter(b):
        pltpu.make_async_copy(
            rows[b], acc_sh.at[di_h[0].at[0]], ssem.at[b]).wait()

    pltpu.sync_copy(src2d_hbm.at[pl.ds(cbase, NB)], si_h[0])
    pltpu.sync_copy(dst2d_hbm.at[pl.ds(cbase, NB)], di_h[0])
    pltpu.async_copy(src2d_hbm.at[pl.ds(cbase + NB, NB)], si_h[1],
                     sisem.at[1])
    pltpu.async_copy(dst2d_hbm.at[pl.ds(cbase + NB, NB)], di_h[1],
                     disem.at[1])
    for k in range(min(GLEAD, NB - 2)):
        fire_gather(k, 0, k)

    @pl.when(c == 0)
    def _():
        descs = [None, None]
        for j in range(2):
            descs[j] = pltpu.async_copy(
                xs_hbm.at[pl.ds(row0 + j * CHUNK, CHUNK)],
                rows[NB - 2 + j], gsem.at[NB - 2 + j])
        for j in range(NWB):
            b = NB - 2 + (j % 2)
            descs[j % 2].wait()
            pltpu.sync_copy(
                rows[b], acc_sh.at[pl.ds(row0 + j * CHUNK, CHUNK)])
            if j + 2 < NWB:
                descs[j % 2] = pltpu.async_copy(
                    xs_hbm.at[pl.ds(row0 + (j + 2) * CHUNK, CHUNK)],
                    rows[b], gsem.at[b])

    @pl.when(c == 1)
    def _():
        def zero_row(i, _):
            def zero_seg(j, _):
                rows[NB - 2][i, pl.ds(j * 16, 16)] = jnp.zeros(
                    (16,), jnp.float32)
                return 0
            lax.fori_loop(0, D_IN // 16, zero_seg, 0)
            return 0

        lax.fori_loop(0, CHUNK, zero_row, 0)
        for j in range(NWB):
            pltpu.sync_copy(
                rows[NB - 2], acc_sh.at[pl.ds(row0 + j * CHUNK, CHUNK)])

    for k in range(NB - 2, GLEAD):
        fire_gather(k, 0, k)

    plsc.subcore_barrier()

    def group_body(g, hc, first=False, last=False):
        hn = 1 - hc
        for b in range(NB):
            if b == 0 and not first:
                pltpu.make_async_copy(
                    dst2d_hbm.at[pl.ds(cbase, NB)], di_h[hc], disem.at[hc]
                ).wait()
            if b == 0 and not first and not last:
                pltpu.async_copy(
                    src2d_hbm.at[pl.ds(cbase + (g + 1) * NB, NB)],
                    si_h[hn], sisem.at[hn])
            wait_gather(b)
            fire_scatter(b, hc)
            if not (first and b < NB - GLEAD):
                wait_scatter((b + GLEAD) % NB)
            if b == 1 and not first and not last:
                pltpu.async_copy(
                    dst2d_hbm.at[pl.ds(cbase + (g + 1) * NB, NB)],
                    di_h[hn], disem.at[hn])
            if b == NB - GLEAD and not last:
                pltpu.make_async_copy(
                    src2d_hbm.at[pl.ds(cbase, NB)], si_h[hn], sisem.at[hn]
                ).wait()
            if not last or b < NB - GLEAD:
                tgt_half = hc if b < NB - GLEAD else hn
                fire_gather((b + GLEAD) % NB, tgt_half, (b + GLEAD) % NB)
        if last:
            for b in range(GLEAD, NB):
                wait_scatter(b)
        return None

    group_body(0, 0, first=True)
    group_body(1, 1)

    def pair(gp, _):
        group_body(gp * 2, 0)
        group_body(gp * 2 + 1, 1)
        return 0

    lax.fori_loop(1, NCH // NB // 2 - 1, pair, 0)

    group_body(NCH // NB - 2, 0)
    group_body(NCH // NB - 1, 1, last=True)

    plsc.subcore_barrier()

    for j in range(NWB):
        b = j % NB
        if j >= NB:
            pltpu.make_async_copy(
                rows[b], out_hbm.at[c, pl.ds(row0 + (j - NB) * CHUNK, CHUNK)],
                gsem.at[b]).wait()
        sl = pl.ds(row0 + j * CHUNK, CHUNK)
        pltpu.sync_copy(acc_sh.at[sl], rows[b])
        pltpu.async_copy(rows[b], out_hbm.at[c, sl], gsem.at[b])
    for j in range(max(0, NWB - NB), NWB):
        b = j % NB
        pltpu.make_async_copy(
            rows[b], out_hbm.at[c, pl.ds(row0 + j * CHUNK, CHUNK)],
            gsem.at[b]).wait()


_BM = 640
_BS = 400


def _scale_body(deg_ref, x_ref, xs_ref, dinv_ref):
    d = deg_ref[...]
    dinv = lax.rsqrt(d[0] + d[1] + 1.0)
    xs_ref[...] = x_ref[...] * dinv
    dinv_ref[...] = dinv


def _scale(deg3, x):
    return pl.pallas_call(
        _scale_body,
        grid=(N // _BS,),
        in_specs=[
            pl.BlockSpec((2, _BS, 1), lambda i: (0, i, 0)),
            pl.BlockSpec((_BS, D_IN), lambda i: (i, 0)),
        ],
        out_specs=[
            pl.BlockSpec((_BS, D_IN), lambda i: (i, 0)),
            pl.BlockSpec((_BS, 1), lambda i: (i, 0)),
        ],
        out_shape=[
            jax.ShapeDtypeStruct((N_PAD, D_IN), jnp.float32),
            jax.ShapeDtypeStruct((N_PAD, 1), jnp.float32),
        ],
    )(deg3, x)


def _dense_body(agg_ref, dinv_ref, w1_ref, b1_ref, w2_ref, out_ref):
    a = agg_ref[...]
    dinv = dinv_ref[...]
    y = (a[0] + a[1]) * dinv
    h = jnp.maximum(
        jnp.dot(y, w1_ref[...], preferred_element_type=jnp.float32)
        + b1_ref[...], 0.0)
    g = jnp.dot(h, w2_ref[...], preferred_element_type=jnp.float32)
    out_ref[...] = g * dinv


def _dense(agg, dinv, W1, b1, W2):
    return pl.pallas_call(
        _dense_body,
        grid=(N // _BS,),
        in_specs=[
            pl.BlockSpec((2, _BS, D_IN), lambda i: (0, i, 0)),
            pl.BlockSpec((_BS, 1), lambda i: (i, 0)),
            pl.BlockSpec((D_IN, D_HID), lambda i: (0, 0)),
            pl.BlockSpec((1, D_HID), lambda i: (0, 0)),
            pl.BlockSpec((D_HID, D_OUT), lambda i: (0, 0)),
        ],
        out_specs=pl.BlockSpec((_BS, D_OUT), lambda i: (i, 0)),
        out_shape=jax.ShapeDtypeStruct((N_PAD, D_OUT), jnp.float32),
    )(agg, dinv, W1, b1, W2)


def _final_body(agg_ref, dinv_ref, b2_ref, out_ref):
    a = agg_ref[...]
    out_ref[...] = (a[0] + a[1]) * dinv_ref[...] + b2_ref[...]


def _final(agg, dinv, b2):
    return pl.pallas_call(
        _final_body,
        grid=(N // _BS,),
        in_specs=[
            pl.BlockSpec((2, _BS, D_OUT), lambda i: (0, i, 0)),
            pl.BlockSpec((_BS, 1), lambda i: (i, 0)),
            pl.BlockSpec((1, D_OUT), lambda i: (0, 0)),
        ],
        out_specs=pl.BlockSpec((_BS, D_OUT), lambda i: (i, 0)),
        out_shape=jax.ShapeDtypeStruct((N, D_OUT), jnp.float32),
    )(agg, dinv, b2)


def kernel(x, edge_index, W1, b1, W2, b2):
    ei = edge_index.astype(jnp.int32)
    n_dummy = E_PAD - E
    pad_idx = N + (jnp.arange(n_dummy, dtype=jnp.int32) % (N_PAD - N))
    src_p = jnp.concatenate([ei[0], pad_idx])
    dst_p = jnp.concatenate([ei[1], pad_idx])
    src2d = src_p.reshape(E_PAD // CHUNK, CHUNK)
    dst2d = dst_p.reshape(E_PAD // CHUNK, CHUNK)
    dst2d_deg = dst_p.reshape(E_PAD // DCHUNK, DCHUNK)

    deg = _deg_kernel(dst2d_deg)
    deg3 = deg.reshape(2, N_PAD, 1)
    xs, dinv = _scale(deg3, x)
    agg1 = _prop_kernel(xs, src2d, dst2d)
    gs = _dense(agg1, dinv, W1, b1.reshape(1, D_HID), W2)
    agg2 = _prop_kernel(gs, src2d, dst2d)
    return _final(agg2, dinv, b2.reshape(1, D_OUT))
